# Initial kernel scaffold; baseline (speedup 1.0000x reference)
import jax
import jax.numpy as jnp
from jax.experimental import pallas as pl


def kernel(x, w):
    raise NotImplementedError("write your pallas kernel here")


# speedup vs baseline: 1.5303x; 1.5303x over previous
import functools
from itertools import combinations

import jax
import jax.numpy as jnp
from jax import lax
from jax.experimental import pallas as pl
from jax.experimental.pallas import tpu as pltpu
from jax.experimental.pallas import tpu_sc as plsc

_F = 5
_D = 128
_L = 16
_NC, _NS = 2, 16
_NW = _NC * _NS
_TRIPLES = list(combinations(range(_F), 3))
_TIDX = {t: i for i, t in enumerate(_TRIPLES)}


def _row_value(vecs, wv):
    acc = None
    for i in range(_F - 2):
        ti = None
        for j in range(i + 1, _F - 1):
            sij = None
            for k in range(j + 1, _F):
                term = wv[_TIDX[(i, j, k)]] * vecs[k]
                sij = term if sij is None else sij + term
            tj = vecs[j] * sij
            ti = tj if ti is None else ti + tj
        tv = vecs[i] * ti
        acc = tv if acc is None else acc + tv
    return acc


def _sc_body(rows_per_w, chunk_rows, x_hbm, w_hbm, out_hbm, buf, wbuf, tbuf,
             obuf):
    wid = lax.axis_index("s") * _NC + lax.axis_index("c")
    base = wid * rows_per_w

    pltpu.sync_copy(w_hbm, wbuf)
    wv = [wbuf[t, :] for t in range(len(_TRIPLES))]
    lane = lax.iota(jnp.int32, _L)

    n_chunks = rows_per_w // chunk_rows
    for chunk in range(n_chunks):
        pltpu.sync_copy(x_hbm.at[pl.ds(base + chunk * chunk_rows, chunk_rows)],
                        buf)

        @pl.loop(0, chunk_rows // _L)
        def _group(g):
            for r16 in range(_L):
                r = g * _L + r16
                acc = None
                for c in range(_D // _L):
                    vecs = [buf[r, pl.ds(f * _D + c * _L, _L)]
                            for f in range(_F)]
                    v = _row_value(vecs, wv)
                    acc = v if acc is None else acc + v
                tbuf[pl.ds(r16 * _L, _L)] = acc
            s = None
            for l in range(_L):
                col = plsc.load_gather(tbuf, [lane * _L + l])
                s = col if s is None else s + col
            obuf[pl.ds(chunk * chunk_rows + g * _L, _L)] = s

    pltpu.sync_copy(obuf, out_hbm.at[pl.ds(base, rows_per_w)])


def kernel(x, w):
    B = x.shape[0]
    rows_per_w = B // _NW
    chunk_rows = 128

    x2 = x.reshape(B, _F * _D)
    wsplat = jnp.broadcast_to(w[:, None], (w.shape[0], _L))

    mesh = plsc.VectorSubcoreMesh(core_axis_name="c", subcore_axis_name="s",
                                  num_cores=_NC, num_subcores=_NS)
    out = pl.kernel(
        functools.partial(_sc_body, rows_per_w, chunk_rows),
        out_type=jax.ShapeDtypeStruct((B,), jnp.float32),
        mesh=mesh,
        compiler_params=pltpu.CompilerParams(needs_layout_passes=False),
        scratch_types=[
            pltpu.VMEM((chunk_rows, _F * _D), jnp.float32),
            pltpu.VMEM((len(_TRIPLES), _L), jnp.float32),
            pltpu.VMEM((_L * _L,), jnp.float32),
            pltpu.VMEM((rows_per_w,), jnp.float32),
        ],
    )(x2, wsplat)
    return out.reshape(B, 1)
